# baseline (speedup 1.0000x reference)
import functools

import jax
import jax.numpy as jnp
from jax import lax
from jax.experimental import pallas as pl
from jax.experimental.pallas import tpu as pltpu
from jax.experimental.pallas import tpu_sc as plsc

NC = 2
NS = 16
NW = NC * NS
L = 16
D = 16
CH = 128


def kernel(x_user, x_movie, edge_label_index, predict_type=0):
    E = edge_label_index.shape[1]
    bpw = E // NW
    nch = bpw // CH
    nb = bpw // L

    idx_u = edge_label_index[0]
    idx_m = edge_label_index[1]

    mesh = plsc.VectorSubcoreMesh(core_axis_name="c", subcore_axis_name="s")

    @functools.partial(
        pl.kernel,
        mesh=mesh,
        out_type=jax.ShapeDtypeStruct((E,), jnp.float32),
        compiler_params=pltpu.CompilerParams(
            needs_layout_passes=False, use_tc_tiling_on_sc=False),
        scratch_types=[
            pltpu.VMEM((bpw,), jnp.int32),
            pltpu.VMEM((bpw,), jnp.int32),
            pltpu.VMEM((bpw, D), jnp.float32),
            pltpu.VMEM((bpw, D), jnp.float32),
            pltpu.VMEM((bpw,), jnp.float32),
            pltpu.SemaphoreType.DMA,
        ],
    )
    def sc_kernel(xu_hbm, xm_hbm, iu_hbm, im_hbm, out_hbm,
                  iu_v, im_v, u_v, m_v, o_v, sem):
        wid = lax.axis_index("s") * NC + lax.axis_index("c")
        base = wid * bpw

        pltpu.sync_copy(iu_hbm.at[pl.ds(base, bpw)], iu_v)
        pltpu.sync_copy(im_hbm.at[pl.ds(base, bpw)], im_v)

        copies = []
        for j in range(nch):
            sl = pl.ds(j * CH, CH)
            copies.append(
                pltpu.async_copy(xu_hbm.at[iu_v.at[sl]], u_v.at[sl], sem))
            copies.append(
                pltpu.async_copy(xm_hbm.at[im_v.at[sl]], m_v.at[sl], sem))
        for c in copies:
            c.wait()

        def body(b, _):
            e0 = b * L
            e_vec = lax.iota(jnp.int32, L) + e0
            acc = jnp.zeros((L,), jnp.float32)
            for d in range(D):
                d_vec = jnp.full((L,), d, jnp.int32)
                uu = plsc.load_gather(u_v, [e_vec, d_vec])
                mm = plsc.load_gather(m_v, [e_vec, d_vec])
                acc = acc + uu * mm
            o_v[pl.ds(e0, L)] = acc
            return 0

        lax.fori_loop(0, nb, body, 0)

        pltpu.sync_copy(o_v, out_hbm.at[pl.ds(base, bpw)])

    return sc_kernel(x_user, x_movie, idx_u, idx_m)


# speedup vs baseline: 5.9623x; 5.9623x over previous
import functools

import jax
import jax.numpy as jnp
from jax import lax
from jax.experimental import pallas as pl
from jax.experimental.pallas import tpu as pltpu
from jax.experimental.pallas import tpu_sc as plsc

NC = 2
NS = 16
NW = NC * NS
L = 16
D = 16
W = 128
C = 8


def kernel(x_user, x_movie, edge_label_index, predict_type=0):
    E = edge_label_index.shape[1]
    bpw = E // NW
    nch = bpw // C

    idx_u = edge_label_index[0]
    idx_m = edge_label_index[1]
    xu_t = x_user.T
    xm_t = x_movie.T

    mesh = plsc.VectorSubcoreMesh(core_axis_name="c", subcore_axis_name="s")

    @functools.partial(
        pl.kernel,
        mesh=mesh,
        out_type=jax.ShapeDtypeStruct((E,), jnp.float32),
        compiler_params=pltpu.CompilerParams(needs_layout_passes=False),
        scratch_types=[
            pltpu.VMEM((bpw,), jnp.int32),
            pltpu.VMEM((bpw,), jnp.int32),
            pltpu.VMEM((2, C, D, W), jnp.float32),
            pltpu.VMEM((2, C, D, W), jnp.float32),
            pltpu.VMEM((L, L), jnp.float32),
            pltpu.VMEM((bpw,), jnp.float32),
            pltpu.SemaphoreType.DMA,
            pltpu.SemaphoreType.DMA,
        ],
    )
    def sc_kernel(xu_hbm, xm_hbm, iu_hbm, im_hbm, out_hbm,
                  iu_s, im_s, bu_v, bm_v, p_v, o_v, sem0, sem1):
        wid = lax.axis_index("s") * NC + lax.axis_index("c")
        base = wid * bpw

        pltpu.sync_copy(iu_hbm.at[pl.ds(base, bpw)], iu_s)
        pltpu.sync_copy(im_hbm.at[pl.ds(base, bpw)], im_s)

        def fire(vecbase, par, slot, sem):
            uvec = iu_s[pl.ds(vecbase, L)]
            mvec = im_s[pl.ds(vecbase, L)]
            for j in range(C):
                for vec, bv, xv in ((uvec, bu_v, xu_hbm), (mvec, bm_v, xm_hbm)):
                    i = vec[par * C + j]
                    a = pl.multiple_of((i >> 7) << 7, W)
                    pltpu.async_copy(xv.at[:, pl.ds(a, W)], bv.at[slot, j], sem)

        def drain(slot, sem):
            for j in range(C):
                for bv, xv in ((bu_v, xu_hbm), (bm_v, xm_hbm)):
                    pltpu.make_async_copy(
                        xv.at[:, pl.ds(0, W)], bv.at[slot, j], sem).wait()

        def compute(vecbase, par, slot, prow):
            iota = lax.iota(jnp.int32, L)
            uvec = iu_s[pl.ds(vecbase, L)] & (W - 1)
            mvec = im_s[pl.ds(vecbase, L)] & (W - 1)
            for j in range(C):
                lu = jnp.full((L,), uvec[par * C + j], jnp.int32)
                lm = jnp.full((L,), mvec[par * C + j], jnp.int32)
                ucol = plsc.load_gather(bu_v.at[slot, j], [iota, lu])
                mcol = plsc.load_gather(bm_v.at[slot, j], [iota, lm])
                p_v[prow + j] = ucol * mcol

        fire(0, 0, 0, sem0)

        def body(h, _):
            c0 = h * 2
            vb = h * L
            fire(vb, 1, 1, sem1)
            drain(0, sem0)
            compute(vb, 0, 0, 0)

            @pl.when(c0 + 2 < nch)
            def _():
                fire(vb + L, 0, 0, sem0)

            drain(1, sem1)
            compute(vb, 1, 1, C)

            iota = lax.iota(jnp.int32, L)
            acc = jnp.zeros((L,), jnp.float32)
            for d in range(D):
                acc = acc + plsc.load_gather(p_v, [iota, jnp.full((L,), d, jnp.int32)])
            o_v[pl.ds(h * L, L)] = acc
            return 0

        lax.fori_loop(0, nch // 2, body, 0)

        pltpu.sync_copy(o_v, out_hbm.at[pl.ds(base, bpw)])

    return sc_kernel(xu_t, xm_t, idx_u, idx_m)


# speedup vs baseline: 6.4917x; 1.0888x over previous
import functools

import jax
import jax.numpy as jnp
from jax import lax
from jax.experimental import pallas as pl
from jax.experimental.pallas import tpu as pltpu
from jax.experimental.pallas import tpu_sc as plsc

NC = 2
NS = 16
NW = NC * NS
L = 16
D = 16
W = 128
C = 4
NSLOT = 4


def kernel(x_user, x_movie, edge_label_index, predict_type=0):
    E = edge_label_index.shape[1]
    bpw = E // NW
    nch = bpw // C

    idx_u = edge_label_index[0]
    idx_m = edge_label_index[1]
    xu_t = x_user.T
    xm_t = x_movie.T

    mesh = plsc.VectorSubcoreMesh(core_axis_name="c", subcore_axis_name="s")

    @functools.partial(
        pl.kernel,
        mesh=mesh,
        out_type=jax.ShapeDtypeStruct((E,), jnp.float32),
        compiler_params=pltpu.CompilerParams(needs_layout_passes=False),
        scratch_types=[
            pltpu.VMEM((bpw,), jnp.int32),
            pltpu.VMEM((bpw,), jnp.int32),
            pltpu.VMEM((NSLOT, C, 2, 8, W), jnp.float32),
            pltpu.VMEM((NSLOT, C, 2, 8, W), jnp.float32),
            pltpu.VMEM((L, L), jnp.float32),
            pltpu.VMEM((bpw,), jnp.float32),
            [pltpu.SemaphoreType.DMA] * NSLOT,
        ],
    )
    def sc_kernel(xu_hbm, xm_hbm, iu_hbm, im_hbm, out_hbm,
                  iu_s, im_s, bu_v, bm_v, p_v, o_v, sems):
        wid = lax.axis_index("s") * NC + lax.axis_index("c")
        base = wid * bpw

        pltpu.sync_copy(iu_hbm.at[pl.ds(base, bpw)], iu_s)
        pltpu.sync_copy(im_hbm.at[pl.ds(base, bpw)], im_s)

        xu3 = xu_hbm.reshape(2, 8, xu_hbm.shape[1])
        xm3 = xm_hbm.reshape(2, 8, xm_hbm.shape[1])

        def fire(vecbase, par, slot):
            uvec = iu_s[pl.ds(vecbase, L)]
            mvec = im_s[pl.ds(vecbase, L)]
            for j in range(C):
                for vec, bv, xv in ((uvec, bu_v, xu3), (mvec, bm_v, xm3)):
                    i = vec[par * C + j]
                    a = pl.multiple_of((i >> 7) << 7, W)
                    for tr in range(2):
                        pltpu.async_copy(xv.at[tr, :, pl.ds(a, W)],
                                         bv.at[slot, j, tr], sems[slot])

        def drain(slot):
            for j in range(C):
                for bv, xv in ((bu_v, xu3), (bm_v, xm3)):
                    pltpu.make_async_copy(
                        xv.at[0, :, pl.ds(0, W)], bv.at[slot, j, 0], sems[slot]).wait()
                    pltpu.make_async_copy(
                        xv.at[1, :, pl.ds(0, W)], bv.at[slot, j, 1], sems[slot]).wait()

        def compute(vecbase, par, slot):
            iota = lax.iota(jnp.int32, L)
            tr = iota >> 3
            sub = iota & 7
            uvec = iu_s[pl.ds(vecbase, L)] & (W - 1)
            mvec = im_s[pl.ds(vecbase, L)] & (W - 1)
            for j in range(C):
                lu = jnp.full((L,), uvec[par * C + j], jnp.int32)
                lm = jnp.full((L,), mvec[par * C + j], jnp.int32)
                ucol = plsc.load_gather(bu_v.at[slot, j], [tr, sub, lu])
                mcol = plsc.load_gather(bm_v.at[slot, j], [tr, sub, lm])
                p_v[par * C + j] = ucol * mcol

        for s in range(NSLOT - 1):
            fire(0, s, s)

        def body(h, _):
            vb = h * L
            c0 = h * NSLOT
            for s in range(NSLOT):
                nxt = c0 + s + (NSLOT - 1)
                npar = (s + NSLOT - 1) % NSLOT
                nvb = vb + (L if s > 0 else 0)

                nslot = (s + NSLOT - 1) % NSLOT

                @pl.when(nxt < nch)
                def _():
                    fire(nvb, npar, nslot)

                drain(s)
                compute(vb, s, s)

            iota = lax.iota(jnp.int32, L)
            acc = jnp.zeros((L,), jnp.float32)
            for d in range(D):
                acc = acc + plsc.load_gather(p_v, [iota, jnp.full((L,), d, jnp.int32)])
            o_v[pl.ds(h * L, L)] = acc
            return 0

        lax.fori_loop(0, nch // NSLOT, body, 0)

        pltpu.sync_copy(o_v, out_hbm.at[pl.ds(base, bpw)])

    return sc_kernel(xu_t, xm_t, idx_u, idx_m)
